# baseline (speedup 1.0000x reference)
import functools

import jax
import jax.numpy as jnp
from jax import lax
from jax.experimental import pallas as pl
from jax.experimental.pallas import tpu as pltpu
from jax.experimental.pallas import tpu_sc as plsc

_B = 16384
_C = 1000
_ROWS = 512
_NEG = -3.0e38


_NC = 2
_NS = 16
_NW = _NC * _NS
_RPW = _B // _NW
_CH = 32
_NCHUNK = _RPW // _CH


def _sc_margin_body(o_hbm, t_hbm, out_hbm, tgt_v, buf0, buf1, marg_v,
                    sem0, sem1):
    wid = lax.axis_index("s") * _NC + lax.axis_index("c")
    base = wid * _RPW
    pltpu.sync_copy(t_hbm.at[pl.ds(base, _RPW)], tgt_v)

    lane = lax.iota(jnp.int32, 16)
    neg = jnp.full((16,), _NEG, jnp.float32)
    bufs = (buf0, buf1)
    sems = (sem0, sem1)

    cp = pltpu.async_copy(o_hbm.at[pl.ds(base * _C, _CH * _C)], buf0, sem0)
    for g in range(_NCHUNK):
        nxt = None
        if g + 1 < _NCHUNK:
            nxt = pltpu.async_copy(
                o_hbm.at[pl.ds((base + (g + 1) * _CH) * _C, _CH * _C)],
                bufs[(g + 1) % 2], sems[(g + 1) % 2])
        cp.wait()
        buf = bufs[g % 2]
        for gg in range(_CH // 16):
            rows_base = (gg * 16 + lane) * _C

            def step(i, carry):
                m1v, m2v = carry
                for u in range(4):
                    idx = rows_base + (i * 4 + u)
                    v = plsc.load_gather(buf, [idx])
                    m2v = jnp.maximum(m2v, jnp.minimum(m1v, v))
                    m1v = jnp.maximum(m1v, v)
                return (m1v, m2v)

            m1v, m2v = lax.fori_loop(0, _C // 4, step, (neg, neg))
            tgt16 = tgt_v[pl.ds(g * _CH + gg * 16, 16)]
            tlv = plsc.load_gather(buf, [rows_base + tgt16])
            margv = jnp.where(m1v == tlv, m1v - m2v, jnp.float32(0.0))
            marg_v[pl.ds(g * _CH + gg * 16, 16)] = margv
        cp = nxt
    pltpu.sync_copy(marg_v, out_hbm.at[pl.ds(base, _RPW)])


@jax.jit
def _sc_margins(o_flat, targets):
    mesh = plsc.VectorSubcoreMesh(core_axis_name="c", subcore_axis_name="s")
    return pl.kernel(
        _sc_margin_body,
        mesh=mesh,
        out_type=jax.ShapeDtypeStruct((_B,), jnp.float32),
        scratch_types=[
            pltpu.VMEM((_RPW,), jnp.int32),
            pltpu.VMEM((_CH * _C,), jnp.float32),
            pltpu.VMEM((_CH * _C,), jnp.float32),
            pltpu.VMEM((_RPW,), jnp.float32),
            pltpu.SemaphoreType.DMA,
            pltpu.SemaphoreType.DMA,
        ],
        compiler_params=pltpu.CompilerParams(needs_layout_passes=False),
    )(o_flat, targets)


def _tc_body(o1, o2, o3, o4, tgt, out, mx):
    t = tgt[:, 0]
    col = jax.lax.broadcasted_iota(jnp.int32, (_ROWS, _C), 1)
    tmask = col == t[:, None]

    def margin(o):
        m1 = jnp.max(o, axis=1)
        tl = jnp.sum(jnp.where(tmask, o, jnp.float32(0.0)), axis=1)
        mx2 = jnp.max(jnp.where(tmask, _NEG, o), axis=1)
        return jnp.where(m1 == tl, m1 - mx2, jnp.float32(0.0)), m1

    d1, x1 = margin(o1[...])
    d2, x2 = margin(o2[...])
    d3, x3 = margin(o3[...])
    d4, x4 = margin(o4[...])
    out[...] = jnp.stack([d1, d2, d3, d4], axis=1)

    bmax = jnp.max(jnp.maximum(jnp.maximum(x1, x2), jnp.maximum(x3, x4)))

    @pl.when(pl.program_id(0) == 0)
    def _():
        mx[...] = bmax[None, None]

    @pl.when(pl.program_id(0) != 0)
    def _():
        mx[...] = jnp.maximum(mx[...], bmax[None, None])


def _combine_body(d14, d5, out):
    preds = jnp.concatenate([d14[...], d5[...]], axis=1) * jnp.float32(0.5)
    preds = preds - jnp.max(preds, axis=1, keepdims=True)
    e = jnp.exp(preds)
    out[...] = e / jnp.sum(e, axis=1, keepdims=True)


@jax.jit
def _run(o1, o2, o3, o4, o5, targets):
    d5 = _sc_margins(o5.reshape(_B * _C), targets)

    grid = (_B // _ROWS,)
    ospec = pl.BlockSpec((_ROWS, _C), lambda i: (i, 0))
    d14, mx = pl.pallas_call(
        _tc_body,
        grid=grid,
        in_specs=[ospec, ospec, ospec, ospec,
                  pl.BlockSpec((_ROWS, 1), lambda i: (i, 0))],
        out_specs=[pl.BlockSpec((_ROWS, 4), lambda i: (i, 0)),
                   pl.BlockSpec((1, 1), lambda i: (0, 0))],
        out_shape=[jax.ShapeDtypeStruct((_B, 4), jnp.float32),
                   jax.ShapeDtypeStruct((1, 1), jnp.float32)],
        compiler_params=pltpu.CompilerParams(
            dimension_semantics=("arbitrary",)),
    )(o1, o2, o3, o4, targets.reshape(_B, 1))

    out = pl.pallas_call(
        _combine_body,
        out_shape=jax.ShapeDtypeStruct((_B, 5), jnp.float32),
    )(d14, d5.reshape(_B, 1))
    return mx[0, 0], out


def kernel(outputs1, outputs2, outputs3, outputs4, mimic, targets, n_test):
    mx, out = _run(outputs1, outputs2, outputs3, outputs4, mimic, targets)
    return mx, out


# speedup vs baseline: 1.1503x; 1.1503x over previous
import functools

import jax
import jax.numpy as jnp
from jax import lax
from jax.experimental import pallas as pl
from jax.experimental.pallas import tpu as pltpu
from jax.experimental.pallas import tpu_sc as plsc

_B = 16384
_C = 1000
_ROWS = 512
_NEG = -3.0e38


_NC = 2
_NS = 16
_NW = _NC * _NS
_RPW = _B // _NW
_CH = 32
_NCHUNK = _RPW // _CH


def _sc_margin_body(o_hbm, t_hbm, out_hbm, tgt_v, buf0, buf1, marg_v,
                    sem0, sem1):
    wid = lax.axis_index("s") * _NC + lax.axis_index("c")
    base = wid * _RPW
    pltpu.sync_copy(t_hbm.at[pl.ds(base, _RPW)], tgt_v)

    lane = lax.iota(jnp.int32, 16)
    neg = jnp.full((16,), _NEG, jnp.float32)
    bufs = (buf0, buf1)
    sems = (sem0, sem1)

    cp = pltpu.async_copy(o_hbm.at[pl.ds(base, _CH), :], buf0, sem0)
    for g in range(_NCHUNK):
        nxt = None
        if g + 1 < _NCHUNK:
            nxt = pltpu.async_copy(
                o_hbm.at[pl.ds(base + (g + 1) * _CH, _CH), :],
                bufs[(g + 1) % 2], sems[(g + 1) % 2])
        cp.wait()
        buf = bufs[g % 2]
        for gg in range(_CH // 16):
            rows = gg * 16 + lane

            def step(i, carry):
                m1v, m2v = carry
                for u in range(4):
                    c = jnp.broadcast_to(i * 4 + u, (16,))
                    v = plsc.load_gather(buf, [rows, c])
                    m2v = jnp.maximum(m2v, jnp.minimum(m1v, v))
                    m1v = jnp.maximum(m1v, v)
                return (m1v, m2v)

            m1v, m2v = lax.fori_loop(0, _C // 4, step, (neg, neg))
            tgt16 = tgt_v[pl.ds(g * _CH + gg * 16, 16)]
            tlv = plsc.load_gather(buf, [rows, tgt16])
            margv = jnp.where(m1v == tlv, m1v - m2v, jnp.float32(0.0))
            marg_v[pl.ds(g * _CH + gg * 16, 16)] = margv
        cp = nxt
    pltpu.sync_copy(marg_v, out_hbm.at[pl.ds(base, _RPW)])


@jax.jit
def _sc_margins(o_flat, targets):
    mesh = plsc.VectorSubcoreMesh(core_axis_name="c", subcore_axis_name="s")
    return pl.kernel(
        _sc_margin_body,
        mesh=mesh,
        out_type=jax.ShapeDtypeStruct((_B,), jnp.float32),
        scratch_types=[
            pltpu.VMEM((_RPW,), jnp.int32),
            pltpu.VMEM((_CH, _C), jnp.float32),
            pltpu.VMEM((_CH, _C), jnp.float32),
            pltpu.VMEM((_RPW,), jnp.float32),
            pltpu.SemaphoreType.DMA,
            pltpu.SemaphoreType.DMA,
        ],
        compiler_params=pltpu.CompilerParams(needs_layout_passes=False),
    )(o_flat, targets)


def _tc_body(o1, o2, o3, o4, tgt, out, mx):
    t = tgt[:, 0]
    col = jax.lax.broadcasted_iota(jnp.int32, (_ROWS, _C), 1)
    tmask = col == t[:, None]

    def margin(o):
        m1 = jnp.max(o, axis=1)
        tl = jnp.sum(jnp.where(tmask, o, jnp.float32(0.0)), axis=1)
        mx2 = jnp.max(jnp.where(tmask, _NEG, o), axis=1)
        return jnp.where(m1 == tl, m1 - mx2, jnp.float32(0.0)), m1

    d1, x1 = margin(o1[...])
    d2, x2 = margin(o2[...])
    d3, x3 = margin(o3[...])
    d4, x4 = margin(o4[...])
    out[...] = jnp.stack([d1, d2, d3, d4], axis=1)

    bmax = jnp.max(jnp.maximum(jnp.maximum(x1, x2), jnp.maximum(x3, x4)))

    @pl.when(pl.program_id(0) == 0)
    def _():
        mx[...] = bmax[None, None]

    @pl.when(pl.program_id(0) != 0)
    def _():
        mx[...] = jnp.maximum(mx[...], bmax[None, None])


def _combine_body(d14, d5, out):
    preds = jnp.concatenate([d14[...], d5[...]], axis=1) * jnp.float32(0.5)
    preds = preds - jnp.max(preds, axis=1, keepdims=True)
    e = jnp.exp(preds)
    out[...] = e / jnp.sum(e, axis=1, keepdims=True)


@jax.jit
def _run(o1, o2, o3, o4, o5, targets):
    d5 = _sc_margins(o5, targets)

    grid = (_B // _ROWS,)
    ospec = pl.BlockSpec((_ROWS, _C), lambda i: (i, 0))
    d14, mx = pl.pallas_call(
        _tc_body,
        grid=grid,
        in_specs=[ospec, ospec, ospec, ospec,
                  pl.BlockSpec((_ROWS, 1), lambda i: (i, 0))],
        out_specs=[pl.BlockSpec((_ROWS, 4), lambda i: (i, 0)),
                   pl.BlockSpec((1, 1), lambda i: (0, 0))],
        out_shape=[jax.ShapeDtypeStruct((_B, 4), jnp.float32),
                   jax.ShapeDtypeStruct((1, 1), jnp.float32)],
        compiler_params=pltpu.CompilerParams(
            dimension_semantics=("arbitrary",)),
    )(o1, o2, o3, o4, targets.reshape(_B, 1))

    out = pl.pallas_call(
        _combine_body,
        out_shape=jax.ShapeDtypeStruct((_B, 5), jnp.float32),
    )(d14, d5.reshape(_B, 1))
    return mx[0, 0], out


def kernel(outputs1, outputs2, outputs3, outputs4, mimic, targets, n_test):
    mx, out = _run(outputs1, outputs2, outputs3, outputs4, mimic, targets)
    return mx, out
